# baseline (speedup 1.0000x reference)
import functools
import math

import jax
import jax.numpy as jnp
from jax import lax
from jax.experimental import pallas as pl
from jax.experimental.pallas import tpu as pltpu
from jax.experimental.pallas import tpu_sc as plsc

_SIZE = 32000
_PAD = 0
_SMOOTH = 0.1
_CONF = 1.0 - _SMOOTH
_EPS = _SMOOTH / (_SIZE - 2)
_ROW_ENT = (_SIZE - 2) * _EPS * math.log(_EPS) + _CONF * math.log(_CONF)

_ROWS = 4096
_RB = 128
_CB = _SIZE

_NC = 2
_NS = 16
_L = 16
_NW = _NC * _NS
_RPW = _ROWS // _NW


def _tc_body(x_ref, tgt_ref, s_ref):
    i = pl.program_id(0)

    @pl.when(i == 0)
    def _init():
        s_ref[0, 0] = 0.0

    xb = x_ref[...]
    tgt = tgt_ref[...]
    nonpad = tgt != _PAD
    gcol = lax.broadcasted_iota(jnp.int32, xb.shape, 1)
    w = jnp.where(nonpad & (gcol != 0), _EPS, 0.0)
    w = jnp.where(nonpad & (gcol == tgt), _CONF, w)
    s_ref[0, 0] += jnp.sum(w * xb)


@functools.partial(
    pl.kernel,
    mesh=plsc.VectorSubcoreMesh(core_axis_name="c", subcore_axis_name="s"),
    out_type=jax.ShapeDtypeStruct((_NW, _L), jnp.float32),
    scratch_types=[
        pltpu.VMEM((_RPW,), jnp.int32),
        pltpu.VMEM((_L,), jnp.float32),
    ],
)
def _sc_count(tgt_hbm, out_hbm, tgt_v, n_stage):
    wid = lax.axis_index("s") * _NC + lax.axis_index("c")
    base = wid * _RPW
    pltpu.sync_copy(tgt_hbm.at[pl.ds(base, _RPW)], tgt_v)
    n_acc = jnp.zeros((_L,), jnp.float32)
    for grp in range(_RPW // _L):
        t16 = tgt_v[pl.ds(grp * _L, _L)]
        n_acc = n_acc + jnp.where(t16 != _PAD, 1.0, 0.0)
    n_stage[...] = n_acc
    pltpu.sync_copy(n_stage, out_hbm.at[wid])


def kernel(x, target):
    tgt_i32 = target.astype(jnp.int32)
    n_parts = _sc_count(tgt_i32)
    s, = pl.pallas_call(
        _tc_body,
        grid=(_ROWS // _RB,),
        in_specs=[
            pl.BlockSpec((_RB, _CB), lambda i: (i, 0)),
            pl.BlockSpec((_RB, 1), lambda i: (i, 0)),
        ],
        out_specs=[
            pl.BlockSpec(memory_space=pltpu.MemorySpace.SMEM),
        ],
        out_shape=[
            jax.ShapeDtypeStruct((1, 1), jnp.float32),
        ],
    )(x, tgt_i32.reshape(_ROWS, 1))
    n = jnp.sum(n_parts)
    return _ROW_ENT * n - s[0, 0]


# speedup vs baseline: 1.0039x; 1.0039x over previous
import functools
import math

import jax
import jax.numpy as jnp
from jax import lax
from jax.experimental import pallas as pl
from jax.experimental.pallas import tpu as pltpu
from jax.experimental.pallas import tpu_sc as plsc

_SIZE = 32000
_PAD = 0
_SMOOTH = 0.1
_CONF = 1.0 - _SMOOTH
_EPS = _SMOOTH / (_SIZE - 2)
_ROW_ENT = (_SIZE - 2) * _EPS * math.log(_EPS) + _CONF * math.log(_CONF)

_ROWS = 4096
_RB = 128
_CB = _SIZE

_NC = 2
_NS = 16
_L = 16
_NW = _NC * _NS
_RPW = _ROWS // _NW


def _tc_body(x_ref, tgt_ref, s_ref):
    i = pl.program_id(0)

    @pl.when(i == 0)
    def _init():
        s_ref[0, 0] = 0.0

    xb = x_ref[...]
    tgt = tgt_ref[...]
    nonpad = tgt != _PAD
    gcol = lax.broadcasted_iota(jnp.int32, xb.shape, 1)
    w = jnp.where(nonpad & (gcol != 0), _EPS, 0.0)
    w = jnp.where(nonpad & (gcol == tgt), _CONF, w)
    s_ref[0, 0] += jnp.sum(w * xb)


@functools.partial(
    pl.kernel,
    mesh=plsc.VectorSubcoreMesh(core_axis_name="c", subcore_axis_name="s"),
    out_type=jax.ShapeDtypeStruct((_NW, _L), jnp.float32),
    scratch_types=[
        pltpu.VMEM((_RPW,), jnp.int32),
        pltpu.VMEM((_L,), jnp.float32),
    ],
)
def _sc_count(tgt_hbm, out_hbm, tgt_v, n_stage):
    wid = lax.axis_index("s") * _NC + lax.axis_index("c")
    base = wid * _RPW
    pltpu.sync_copy(tgt_hbm.at[pl.ds(base, _RPW)], tgt_v)
    n_acc = jnp.zeros((_L,), jnp.float32)
    for grp in range(_RPW // _L):
        t16 = tgt_v[pl.ds(grp * _L, _L)]
        n_acc = n_acc + jnp.where(t16 != _PAD, 1.0, 0.0)
    n_stage[...] = n_acc
    pltpu.sync_copy(n_stage, out_hbm.at[wid])


def kernel(x, target):
    tgt_i32 = target.astype(jnp.int32)
    s, = pl.pallas_call(
        _tc_body,
        grid=(_ROWS // _RB,),
        in_specs=[
            pl.BlockSpec((_RB, _CB), lambda i: (i, 0)),
            pl.BlockSpec((_RB, 1), lambda i: (i, 0)),
        ],
        out_specs=[
            pl.BlockSpec(memory_space=pltpu.MemorySpace.SMEM),
        ],
        out_shape=[
            jax.ShapeDtypeStruct((1, 1), jnp.float32),
        ],
    )(x, tgt_i32.reshape(_ROWS, 1))
    n_parts = _sc_count(tgt_i32)
    n = jnp.sum(n_parts)
    return _ROW_ENT * n - s[0, 0]
